# baseline (speedup 1.0000x reference)
import functools

import jax
import jax.numpy as jnp
from jax import lax
from jax.experimental import pallas as pl
from jax.experimental.pallas import tpu as pltpu
from jax.experimental.pallas import tpu_sc as plsc

_N_PATCHES = 1024
_HIDDEN = 768
_NC = 2
_CHUNKS = 4
_BLK = _N_PATCHES // (_NC * _CHUNKS)


@functools.cache
def _sc_lookup():
    mesh = plsc.ScalarSubcoreMesh(axis_name="c", num_cores=_NC)

    @functools.partial(
        pl.kernel,
        out_type=jax.ShapeDtypeStruct(
            (_NC * _CHUNKS, _BLK, _HIDDEN), jnp.float32),
        mesh=mesh,
        scratch_types=[
            pltpu.SMEM((1,), jnp.int32),
            pltpu.SemaphoreType.DMA,
        ],
    )
    def body(table_hbm, lev_hbm, out_hbm, lev_s, sem):
        cid = lax.axis_index("c")
        pltpu.sync_copy(lev_hbm, lev_s)
        base = (lev_s[0] - 1) * _NC * _CHUNKS + cid * _CHUNKS
        copies = []
        for k in range(_CHUNKS):
            copies.append(pltpu.make_async_copy(
                table_hbm.at[pl.ds(base + k, 1)],
                out_hbm.at[pl.ds(cid * _CHUNKS + k, 1)],
                sem))
            copies[-1].start()
        for c in copies:
            c.wait()

    return body


def kernel(x, lev, table):
    lev32 = jnp.asarray(lev, jnp.int32).reshape(1)
    out = _sc_lookup()(table.reshape(-1, _BLK, _HIDDEN), lev32)
    return out.reshape(1, _N_PATCHES, _HIDDEN)[:, : x.shape[1]]


# speedup vs baseline: 5.1519x; 5.1519x over previous
import functools

import jax
import jax.numpy as jnp
from jax import lax
from jax.experimental import pallas as pl
from jax.experimental.pallas import tpu as pltpu
from jax.experimental.pallas import tpu_sc as plsc

_N_PATCHES = 1024
_HIDDEN = 768
_NC = 2
_NS = 16
_NW = _NC * _NS
_BLOCK = _N_PATCHES // (2 * _NW)


@functools.cache
def _sc_lookup():
    mesh = plsc.VectorSubcoreMesh(core_axis_name="c", subcore_axis_name="s")

    @functools.partial(
        pl.kernel,
        out_type=jax.ShapeDtypeStruct((2 * _NW, _BLOCK, _HIDDEN), jnp.float32),
        mesh=mesh,
        scratch_types=[
            pltpu.VMEM((2, 1), jnp.int32),
            pltpu.VMEM((1, _BLOCK, _HIDDEN), jnp.float32),
            pltpu.VMEM((1, _BLOCK, _HIDDEN), jnp.float32),
            pltpu.SemaphoreType.DMA,
            pltpu.SemaphoreType.DMA,
            pltpu.SemaphoreType.DMA,
        ],
    )
    def body(table_hbm, idx_hbm, out_hbm, idx_v, rows_a, rows_b, sem_a, sem_b,
             sem_st):
        wid = lax.axis_index("s") * _NC + lax.axis_index("c")
        pltpu.sync_copy(idx_hbm.at[wid], idx_v)
        ga = pltpu.async_copy(table_hbm.at[idx_v.at[0]], rows_a, sem_a)
        gb = pltpu.async_copy(table_hbm.at[idx_v.at[1]], rows_b, sem_b)
        ga.wait()
        st = pltpu.async_copy(rows_a, out_hbm.at[pl.ds(2 * wid, 1)], sem_st)
        gb.wait()
        pltpu.sync_copy(rows_b, out_hbm.at[pl.ds(2 * wid + 1, 1)])
        st.wait()

    return body


def kernel(x, lev, table):
    lev32 = jnp.asarray(lev, jnp.int32)
    idx = (lev32 - 1) * 2 * _NW + jnp.arange(2 * _NW, dtype=jnp.int32)
    out = _sc_lookup()(
        table.reshape(-1, _BLOCK, _HIDDEN), idx.reshape(_NW, 2, 1))
    return out.reshape(1, _N_PATCHES, _HIDDEN)[:, : x.shape[1]]
